# baseline (speedup 1.0000x reference)
import jax
import jax.numpy as jnp
from jax import lax
from jax.experimental import pallas as pl
from jax.experimental.pallas import tpu as pltpu
from jax.experimental.pallas import tpu_sc as plsc

_VOCAB = 100000
_HID = 768
_MAXS = 2048
_B = 4
_EPS = 1e-12

_NW = 32
_N = _B * _MAXS
_TPW = _N // _NW
_C = 64
_NCH = _TPW // _C

_BN = 256
_NBLK = _N // _BN
_SPB = _MAXS // _BN


def _sc_gather_body(ids_hbm, tok_hbm, out_hbm,
                    ids0, ids1, buf0, buf1, gsem0, gsem1, wsem0, wsem1):
    wid = lax.axis_index("s") * 2 + lax.axis_index("c")
    base = wid * _TPW
    idbufs = (ids0, ids1)
    bufs = (buf0, buf1)
    gsems = (gsem0, gsem1)
    wsems = (wsem0, wsem1)

    def ids_slice(g):
        n = base + g * _C
        return ids_hbm.at[n // _MAXS, pl.ds(lax.rem(n, _MAXS), _C)]

    pltpu.sync_copy(ids_slice(0), ids0)
    gathers = [pltpu.async_copy(tok_hbm.at[ids0], buf0, gsem0), None]
    writes = [None, None]
    for g in range(_NCH):
        p = g % 2
        np_ = (g + 1) % 2
        if g + 1 < _NCH:
            if writes[np_] is not None:
                writes[np_].wait()
                writes[np_] = None
            pltpu.sync_copy(ids_slice(g + 1), idbufs[np_])
            gathers[np_] = pltpu.async_copy(
                tok_hbm.at[idbufs[np_]], bufs[np_], gsems[np_])
        gathers[p].wait()
        writes[p] = pltpu.async_copy(
            bufs[p], out_hbm.at[pl.ds(base + g * _C, _C)], wsems[p])
    for p in range(2):
        if writes[p] is not None:
            writes[p].wait()


def _tc_ln_body(emb_ref, pos_ref, sid_ref, seg_ref, gam_ref, bet_ref, out_ref):
    g = pl.program_id(0)
    prow = lax.rem(g, _SPB) * _BN
    e = emb_ref[...] + pos_ref[pl.ds(prow, _BN), :]
    sidf = sid_ref[0].astype(jnp.float32)
    s0 = seg_ref[0:1, :]
    s1 = seg_ref[1:2, :]
    e = e + s0 + sidf.reshape(_BN, 1) * (s1 - s0)
    mean = jnp.mean(e, axis=-1, keepdims=True)
    var = jnp.mean((e - mean) ** 2, axis=-1, keepdims=True)
    normed = (e - mean) * lax.rsqrt(var + _EPS)
    out_ref[...] = normed * gam_ref[...].reshape(1, _HID) + bet_ref[...].reshape(1, _HID)


@jax.jit
def _run(ids_w, sid3, token_table, pos_table, seg_table, gamma, beta):
    mesh = plsc.VectorSubcoreMesh(core_axis_name="c", subcore_axis_name="s")
    gath = pl.kernel(
        _sc_gather_body,
        out_type=jax.ShapeDtypeStruct((_N, _HID), jnp.float32),
        mesh=mesh,
        compiler_params=pltpu.CompilerParams(needs_layout_passes=False),
        scratch_types=[
            pltpu.VMEM((_C,), jnp.int32),
            pltpu.VMEM((_C,), jnp.int32),
            pltpu.VMEM((_C, _HID), jnp.float32),
            pltpu.VMEM((_C, _HID), jnp.float32),
            pltpu.SemaphoreType.DMA,
            pltpu.SemaphoreType.DMA,
            pltpu.SemaphoreType.DMA,
            pltpu.SemaphoreType.DMA,
        ],
    )
    rows = gath(ids_w, token_table)

    out = pl.pallas_call(
        _tc_ln_body,
        out_shape=jax.ShapeDtypeStruct((_N, _HID), jnp.float32),
        grid=(_NBLK,),
        in_specs=[
            pl.BlockSpec((_BN, _HID), lambda g: (g, 0)),
            pl.BlockSpec((_MAXS, _HID), lambda g: (0, 0)),
            pl.BlockSpec((1, 1, _BN), lambda g: (g, 0, 0)),
            pl.BlockSpec((2, _HID), lambda g: (0, 0)),
            pl.BlockSpec((_HID,), lambda g: (0,)),
            pl.BlockSpec((_HID,), lambda g: (0,)),
        ],
        out_specs=pl.BlockSpec((_BN, _HID), lambda g: (g, 0)),
    )(rows, pos_table, sid3, seg_table, gamma, beta)
    return out


def kernel(input_ids, segment_ids, token_table, pos_table, seg_table, gamma, beta):
    sid3 = segment_ids.reshape(_NBLK, 1, _BN)
    out = _run(input_ids, sid3, token_table, pos_table, seg_table, gamma, beta)
    return out.reshape(_B, _MAXS, _HID)


# speedup vs baseline: 1.1362x; 1.1362x over previous
import jax
import jax.numpy as jnp
from jax import lax
from jax.experimental import pallas as pl
from jax.experimental.pallas import tpu as pltpu
from jax.experimental.pallas import tpu_sc as plsc

_VOCAB = 100000
_HID = 768
_MAXS = 2048
_B = 4
_EPS = 1e-12

_NW = 32
_N = _B * _MAXS
_TPW = _N // _NW
_C = 64
_NCH = _TPW // _C

_BN = 512
_NBLK = _N // _BN
_SPB = _MAXS // _BN


def _sc_gather_body(ids_hbm, tok_hbm, out_hbm,
                    ids0, ids1, buf0, buf1, gsem0, gsem1, wsem0, wsem1):
    wid = lax.axis_index("s") * 2 + lax.axis_index("c")
    base = wid * _TPW
    idbufs = (ids0, ids1)
    bufs = (buf0, buf1)
    gsems = (gsem0, gsem1)
    wsems = (wsem0, wsem1)

    def ids_slice(g):
        n = base + g * _C
        return ids_hbm.at[n // _MAXS, pl.ds(lax.rem(n, _MAXS), _C)]

    pltpu.sync_copy(ids_slice(0), ids0)
    gathers = [pltpu.async_copy(tok_hbm.at[ids0], buf0, gsem0), None]
    writes = [None, None]
    for g in range(_NCH):
        p = g % 2
        np_ = (g + 1) % 2
        if g + 1 < _NCH:
            if writes[np_] is not None:
                writes[np_].wait()
                writes[np_] = None
            pltpu.sync_copy(ids_slice(g + 1), idbufs[np_])
            gathers[np_] = pltpu.async_copy(
                tok_hbm.at[idbufs[np_]], bufs[np_], gsems[np_])
        gathers[p].wait()
        writes[p] = pltpu.async_copy(
            bufs[p], out_hbm.at[pl.ds(base + g * _C, _C)], wsems[p])
    for p in range(2):
        if writes[p] is not None:
            writes[p].wait()


def _tc_ln_body(emb_ref, pos_ref, sid_ref, seg_ref, gam_ref, bet_ref, out_ref):
    g = pl.program_id(0)
    prow = lax.rem(g, _SPB) * _BN
    e = emb_ref[...] + pos_ref[pl.ds(prow, _BN), :]
    sidf = sid_ref[0].astype(jnp.float32)
    s0 = seg_ref[0:1, :]
    s1 = seg_ref[1:2, :]
    e = e + s0 + sidf.reshape(_BN, 1) * (s1 - s0)
    mean = jnp.mean(e, axis=-1, keepdims=True)
    var = jnp.mean((e - mean) ** 2, axis=-1, keepdims=True)
    normed = (e - mean) * lax.rsqrt(var + _EPS)
    out_ref[...] = normed * gam_ref[...].reshape(1, _HID) + bet_ref[...].reshape(1, _HID)


@jax.jit
def _run(ids_w, sid3, token_table, pos_table, seg_table, gamma, beta):
    mesh = plsc.VectorSubcoreMesh(core_axis_name="c", subcore_axis_name="s")
    gath = pl.kernel(
        _sc_gather_body,
        out_type=jax.ShapeDtypeStruct((_N, _HID), jnp.float32),
        mesh=mesh,
        compiler_params=pltpu.CompilerParams(needs_layout_passes=False),
        scratch_types=[
            pltpu.VMEM((_C,), jnp.int32),
            pltpu.VMEM((_C,), jnp.int32),
            pltpu.VMEM((_C, _HID), jnp.float32),
            pltpu.VMEM((_C, _HID), jnp.float32),
            pltpu.SemaphoreType.DMA,
            pltpu.SemaphoreType.DMA,
            pltpu.SemaphoreType.DMA,
            pltpu.SemaphoreType.DMA,
        ],
    )
    rows = gath(ids_w, token_table)

    out = pl.pallas_call(
        _tc_ln_body,
        out_shape=jax.ShapeDtypeStruct((_N, _HID), jnp.float32),
        grid=(_NBLK,),
        in_specs=[
            pl.BlockSpec((_BN, _HID), lambda g: (g, 0)),
            pl.BlockSpec((_MAXS, _HID), lambda g: (0, 0)),
            pl.BlockSpec((1, 1, _BN), lambda g: (g, 0, 0)),
            pl.BlockSpec((2, _HID), lambda g: (0, 0)),
            pl.BlockSpec((_HID,), lambda g: (0,)),
            pl.BlockSpec((_HID,), lambda g: (0,)),
        ],
        out_specs=pl.BlockSpec((_BN, _HID), lambda g: (g, 0)),
    )(rows, pos_table, sid3, seg_table, gamma, beta)
    return out


def kernel(input_ids, segment_ids, token_table, pos_table, seg_table, gamma, beta):
    sid3 = segment_ids.reshape(_NBLK, 1, _BN)
    out = _run(input_ids, sid3, token_table, pos_table, seg_table, gamma, beta)
    return out.reshape(_B, _MAXS, _HID)


# speedup vs baseline: 1.1685x; 1.0284x over previous
import functools

import jax
import jax.numpy as jnp
from jax import lax
from jax.experimental import pallas as pl
from jax.experimental.pallas import tpu as pltpu
from jax.experimental.pallas import tpu_sc as plsc

_VOCAB = 100000
_HID = 768
_MAXS = 2048
_B = 4
_EPS = 1e-12

_NW = 32
_N = _B * _MAXS
_N2 = _N // 2
_TPW = _N2 // _NW
_C = 64
_NCH = _TPW // _C

_BN = 512
_NBLK = _N // _BN
_NBLK2 = _N2 // _BN
_SPB = _MAXS // _BN


def _sc_gather_body(half, ids_hbm, tok_hbm, out_hbm,
                    ids0, ids1, buf0, buf1, gsem0, gsem1, wsem0, wsem1):
    wid = lax.axis_index("s") * 2 + lax.axis_index("c")
    base = wid * _TPW
    idbufs = (ids0, ids1)
    bufs = (buf0, buf1)
    gsems = (gsem0, gsem1)
    wsems = (wsem0, wsem1)

    def ids_slice(g):
        n = half * _N2 + base + g * _C
        return ids_hbm.at[n // _MAXS, pl.ds(lax.rem(n, _MAXS), _C)]

    pltpu.sync_copy(ids_slice(0), ids0)
    gathers = [pltpu.async_copy(tok_hbm.at[ids0], buf0, gsem0), None]
    writes = [None, None]
    for g in range(_NCH):
        p = g % 2
        np_ = (g + 1) % 2
        if g + 1 < _NCH:
            if writes[np_] is not None:
                writes[np_].wait()
                writes[np_] = None
            pltpu.sync_copy(ids_slice(g + 1), idbufs[np_])
            gathers[np_] = pltpu.async_copy(
                tok_hbm.at[idbufs[np_]], bufs[np_], gsems[np_])
        gathers[p].wait()
        writes[p] = pltpu.async_copy(
            bufs[p], out_hbm.at[pl.ds(base + g * _C, _C)], wsems[p])
    for p in range(2):
        if writes[p] is not None:
            writes[p].wait()


def _ln_block(emb, pos_ref, sid_ref, seg_ref, gam_ref, bet_ref, gblk):
    prow = lax.rem(gblk, _SPB) * _BN
    e = emb + pos_ref[pl.ds(prow, _BN), :]
    sidf = sid_ref[0].astype(jnp.float32)
    s0 = seg_ref[0:1, :]
    s1 = seg_ref[1:2, :]
    e = e + s0 + sidf.reshape(_BN, 1) * (s1 - s0)
    mean = jnp.mean(e, axis=-1, keepdims=True)
    var = jnp.mean((e - mean) ** 2, axis=-1, keepdims=True)
    normed = (e - mean) * lax.rsqrt(var + _EPS)
    return normed * gam_ref[...].reshape(1, _HID) + bet_ref[...].reshape(1, _HID)


def _tc_ln0_body(emb_ref, pos_ref, sid_ref, seg_ref, gam_ref, bet_ref, out_ref):
    g = pl.program_id(0)
    out_ref[...] = _ln_block(emb_ref[...], pos_ref, sid_ref, seg_ref,
                             gam_ref, bet_ref, g)


def _tc_ln1_body(big_ref, emb_ref, pos_ref, sid_ref, seg_ref, gam_ref, bet_ref,
                 out_ref):
    del big_ref
    g = pl.program_id(0)
    out_ref[...] = _ln_block(emb_ref[...], pos_ref, sid_ref, seg_ref,
                             gam_ref, bet_ref, g + _NBLK2)


_COMMON_SPECS = [
    pl.BlockSpec((_MAXS, _HID), lambda g: (0, 0)),
    pl.BlockSpec((1, 1, _BN), lambda g: (g, 0, 0)),
    pl.BlockSpec((2, _HID), lambda g: (0, 0)),
    pl.BlockSpec((_HID,), lambda g: (0,)),
    pl.BlockSpec((_HID,), lambda g: (0,)),
]


@jax.jit
def _run(input_ids, sid3, token_table, pos_table, seg_table, gamma, beta):
    mesh = plsc.VectorSubcoreMesh(core_axis_name="c", subcore_axis_name="s")
    scratch = [
        pltpu.VMEM((_C,), jnp.int32),
        pltpu.VMEM((_C,), jnp.int32),
        pltpu.VMEM((_C, _HID), jnp.float32),
        pltpu.VMEM((_C, _HID), jnp.float32),
        pltpu.SemaphoreType.DMA,
        pltpu.SemaphoreType.DMA,
        pltpu.SemaphoreType.DMA,
        pltpu.SemaphoreType.DMA,
    ]
    cp = pltpu.CompilerParams(needs_layout_passes=False)
    gath0 = pl.kernel(
        functools.partial(_sc_gather_body, 0),
        out_type=jax.ShapeDtypeStruct((_N, _HID), jnp.float32),
        mesh=mesh, compiler_params=cp, scratch_types=scratch,
    )
    gath1 = pl.kernel(
        functools.partial(_sc_gather_body, 1),
        out_type=jax.ShapeDtypeStruct((_N2, _HID), jnp.float32),
        mesh=mesh, compiler_params=cp, scratch_types=scratch,
    )
    big = gath0(input_ids, token_table)
    rows1 = gath1(input_ids, token_table)

    big = pl.pallas_call(
        _tc_ln0_body,
        out_shape=jax.ShapeDtypeStruct((_N, _HID), jnp.float32),
        grid=(_NBLK2,),
        in_specs=[pl.BlockSpec((_BN, _HID), lambda g: (g, 0))] + _COMMON_SPECS,
        out_specs=pl.BlockSpec((_BN, _HID), lambda g: (g, 0)),
        input_output_aliases={0: 0},
    )(big, pos_table, sid3[:_NBLK2], seg_table, gamma, beta)

    out = pl.pallas_call(
        _tc_ln1_body,
        out_shape=jax.ShapeDtypeStruct((_N, _HID), jnp.float32),
        grid=(_NBLK2,),
        in_specs=[pl.BlockSpec((8, 128), lambda g: (0, 0)),
                  pl.BlockSpec((_BN, _HID), lambda g: (g, 0))] + _COMMON_SPECS,
        out_specs=pl.BlockSpec((_BN, _HID), lambda g: (g + _NBLK2, 0)),
        input_output_aliases={0: 0},
    )(big, rows1, pos_table, sid3[_NBLK2:], seg_table, gamma, beta)
    return out


def kernel(input_ids, segment_ids, token_table, pos_table, seg_table, gamma, beta):
    sid3 = segment_ids.reshape(_NBLK, 1, _BN)
    out = _run(input_ids, sid3, token_table, pos_table, seg_table, gamma, beta)
    return out.reshape(_B, _MAXS, _HID)


# speedup vs baseline: 1.1932x; 1.0212x over previous
import functools

import jax
import jax.numpy as jnp
from jax import lax
from jax.experimental import pallas as pl
from jax.experimental.pallas import tpu as pltpu
from jax.experimental.pallas import tpu_sc as plsc

_VOCAB = 100000
_HID = 768
_MAXS = 2048
_B = 4
_EPS = 1e-12

_NW = 32
_N = _B * _MAXS
_N2 = _N // 2
_TPW = _N2 // _NW
_C = 64
_NCH = _TPW // _C

_BN = 1024
_NBLK = _N // _BN
_NBLK2 = _N2 // _BN
_SPB = _MAXS // _BN


def _sc_gather_body(half, ids_hbm, tok_hbm, out_hbm,
                    ids0, ids1, buf0, buf1, gsem0, gsem1, wsem0, wsem1):
    wid = lax.axis_index("s") * 2 + lax.axis_index("c")
    base = wid * _TPW
    idbufs = (ids0, ids1)
    bufs = (buf0, buf1)
    gsems = (gsem0, gsem1)
    wsems = (wsem0, wsem1)

    def ids_slice(g):
        n = half * _N2 + base + g * _C
        return ids_hbm.at[n // _MAXS, pl.ds(lax.rem(n, _MAXS), _C)]

    pltpu.sync_copy(ids_slice(0), ids0)
    gathers = [pltpu.async_copy(tok_hbm.at[ids0], buf0, gsem0), None]
    writes = [None, None]
    for g in range(_NCH):
        p = g % 2
        np_ = (g + 1) % 2
        if g + 1 < _NCH:
            if writes[np_] is not None:
                writes[np_].wait()
                writes[np_] = None
            pltpu.sync_copy(ids_slice(g + 1), idbufs[np_])
            gathers[np_] = pltpu.async_copy(
                tok_hbm.at[idbufs[np_]], bufs[np_], gsems[np_])
        gathers[p].wait()
        writes[p] = pltpu.async_copy(
            bufs[p], out_hbm.at[pl.ds(base + g * _C, _C)], wsems[p])
    for p in range(2):
        if writes[p] is not None:
            writes[p].wait()


def _ln_block(emb, pos_ref, sid_ref, seg_ref, gam_ref, bet_ref, gblk):
    prow = lax.rem(gblk, _SPB) * _BN
    e = emb + pos_ref[pl.ds(prow, _BN), :]
    sidf = sid_ref[0].astype(jnp.float32)
    s0 = seg_ref[0:1, :]
    s1 = seg_ref[1:2, :]
    e = e + s0 + sidf.reshape(_BN, 1) * (s1 - s0)
    mean = jnp.mean(e, axis=-1, keepdims=True)
    var = jnp.mean((e - mean) ** 2, axis=-1, keepdims=True)
    normed = (e - mean) * lax.rsqrt(var + _EPS)
    return normed * gam_ref[...].reshape(1, _HID) + bet_ref[...].reshape(1, _HID)


def _tc_ln0_body(emb_ref, pos_ref, sid_ref, seg_ref, gam_ref, bet_ref, out_ref):
    g = pl.program_id(0)
    out_ref[...] = _ln_block(emb_ref[...], pos_ref, sid_ref, seg_ref,
                             gam_ref, bet_ref, g)


def _tc_ln1_body(big_ref, emb_ref, pos_ref, sid_ref, seg_ref, gam_ref, bet_ref,
                 out_ref):
    del big_ref
    g = pl.program_id(0)
    out_ref[...] = _ln_block(emb_ref[...], pos_ref, sid_ref, seg_ref,
                             gam_ref, bet_ref, g + _NBLK2)


_COMMON_SPECS = [
    pl.BlockSpec((_MAXS, _HID), lambda g: (0, 0)),
    pl.BlockSpec((1, 1, _BN), lambda g: (g, 0, 0)),
    pl.BlockSpec((2, _HID), lambda g: (0, 0)),
    pl.BlockSpec((_HID,), lambda g: (0,)),
    pl.BlockSpec((_HID,), lambda g: (0,)),
]


@jax.jit
def _run(input_ids, sid3, token_table, pos_table, seg_table, gamma, beta):
    mesh = plsc.VectorSubcoreMesh(core_axis_name="c", subcore_axis_name="s")
    scratch = [
        pltpu.VMEM((_C,), jnp.int32),
        pltpu.VMEM((_C,), jnp.int32),
        pltpu.VMEM((_C, _HID), jnp.float32),
        pltpu.VMEM((_C, _HID), jnp.float32),
        pltpu.SemaphoreType.DMA,
        pltpu.SemaphoreType.DMA,
        pltpu.SemaphoreType.DMA,
        pltpu.SemaphoreType.DMA,
    ]
    cp = pltpu.CompilerParams(needs_layout_passes=False)
    gath0 = pl.kernel(
        functools.partial(_sc_gather_body, 0),
        out_type=jax.ShapeDtypeStruct((_N, _HID), jnp.float32),
        mesh=mesh, compiler_params=cp, scratch_types=scratch,
    )
    gath1 = pl.kernel(
        functools.partial(_sc_gather_body, 1),
        out_type=jax.ShapeDtypeStruct((_N2, _HID), jnp.float32),
        mesh=mesh, compiler_params=cp, scratch_types=scratch,
    )
    big = gath0(input_ids, token_table)
    rows1 = gath1(input_ids, token_table)

    big = pl.pallas_call(
        _tc_ln0_body,
        out_shape=jax.ShapeDtypeStruct((_N, _HID), jnp.float32),
        grid=(_NBLK2,),
        in_specs=[pl.BlockSpec((_BN, _HID), lambda g: (g, 0))] + _COMMON_SPECS,
        out_specs=pl.BlockSpec((_BN, _HID), lambda g: (g, 0)),
        input_output_aliases={0: 0},
    )(big, pos_table, sid3[:_NBLK2], seg_table, gamma, beta)

    out = pl.pallas_call(
        _tc_ln1_body,
        out_shape=jax.ShapeDtypeStruct((_N, _HID), jnp.float32),
        grid=(_NBLK2,),
        in_specs=[pl.BlockSpec((8, 128), lambda g: (0, 0)),
                  pl.BlockSpec((_BN, _HID), lambda g: (g, 0))] + _COMMON_SPECS,
        out_specs=pl.BlockSpec((_BN, _HID), lambda g: (g + _NBLK2, 0)),
        input_output_aliases={0: 0},
    )(big, rows1, pos_table, sid3[_NBLK2:], seg_table, gamma, beta)
    return out


def kernel(input_ids, segment_ids, token_table, pos_table, seg_table, gamma, beta):
    sid3 = segment_ids.reshape(_NBLK, 1, _BN)
    out = _run(input_ids, sid3, token_table, pos_table, seg_table, gamma, beta)
    return out.reshape(_B, _MAXS, _HID)
